# Initial kernel scaffold; baseline (speedup 1.0000x reference)
import jax
import jax.numpy as jnp
from jax.experimental import pallas as pl


def kernel(attn_bias, spatial_pos, W):
    raise NotImplementedError("write your pallas kernel here")


# speedup vs baseline: 21.0182x; 21.0182x over previous
import functools

import jax
import jax.numpy as jnp
from jax import lax
from jax.experimental import pallas as pl
from jax.experimental.pallas import tpu as pltpu
from jax.experimental.pallas import tpu_sc as plsc

NUM_HEADS = 16
N = 2048
NN = N * N

_info = plsc.get_sparse_core_info()
_NC, _NS, _L = _info.num_cores, _info.num_subcores, _info.num_lanes
_NW = _NC * _NS
_B_PER_W = NN // _NW
_CHUNK = 4096
_N_CHUNKS = _B_PER_W // _CHUNK


def _sc_gather(idx_flat, table):
    mesh = plsc.VectorSubcoreMesh(core_axis_name="c", subcore_axis_name="s")

    @functools.partial(
        pl.kernel,
        mesh=mesh,
        compiler_params=pltpu.CompilerParams(use_tc_tiling_on_sc=False),
        out_type=jax.ShapeDtypeStruct((NN, NUM_HEADS), jnp.float32),
        scratch_types=[
            pltpu.VMEM((_CHUNK,), jnp.int32),
            pltpu.VMEM((_CHUNK, NUM_HEADS), jnp.float32),
            pltpu.SemaphoreType.DMA,
        ],
    )
    def k(table_hbm, idx_hbm, out_hbm, idx_v, rows_v, sem):
        wid = lax.axis_index("s") * _NC + lax.axis_index("c")
        base = wid * _B_PER_W

        def body(t, carry):
            off = base + t * _CHUNK
            pltpu.sync_copy(idx_hbm.at[pl.ds(off, _CHUNK)], idx_v)
            pltpu.async_copy(table_hbm.at[idx_v], rows_v, sem).wait()
            pltpu.sync_copy(rows_v, out_hbm.at[pl.ds(off, _CHUNK)])
            return carry

        lax.fori_loop(0, _N_CHUNKS, body, 0)

    return k(table, idx_flat)


_BT = 4096


def _tc_body(g_ref, b_ref, out_ref):
    gb = g_ref[0]
    ident = (
        lax.broadcasted_iota(jnp.int32, (NUM_HEADS, NUM_HEADS), 0)
        == lax.broadcasted_iota(jnp.int32, (NUM_HEADS, NUM_HEADS), 1)
    ).astype(jnp.float32)
    t = lax.dot_general(
        ident, gb, (((1,), (1,)), ((), ())),
        preferred_element_type=jnp.float32,
    )
    out_ref[...] = t + b_ref[0]


def _tc_transpose_add(g, bias_flat):
    grid = NN // _BT
    g3 = g.reshape(grid, _BT, NUM_HEADS)
    b3 = bias_flat.reshape(grid, 1, _BT)
    return pl.pallas_call(
        _tc_body,
        grid=(grid,),
        in_specs=[
            pl.BlockSpec((1, _BT, NUM_HEADS), lambda i: (i, 0, 0)),
            pl.BlockSpec((1, 1, _BT), lambda i: (i, 0, 0)),
        ],
        out_specs=pl.BlockSpec((NUM_HEADS, _BT), lambda i: (0, i)),
        out_shape=jax.ShapeDtypeStruct((NUM_HEADS, NN), jnp.float32),
    )(g3, b3)


def kernel(attn_bias, spatial_pos, W):
    idx_flat = spatial_pos.reshape(NN)
    g = _sc_gather(idx_flat, W)
    out = _tc_transpose_add(g, attn_bias.reshape(NN))
    return out.reshape(1, NUM_HEADS, N, N)


# speedup vs baseline: 30.2425x; 1.4389x over previous
import functools

import jax
import jax.numpy as jnp
from jax import lax
from jax.experimental import pallas as pl
from jax.experimental.pallas import tpu as pltpu
from jax.experimental.pallas import tpu_sc as plsc

NUM_HEADS = 16
N = 2048
NN = N * N

_info = plsc.get_sparse_core_info()
_NC, _NS, _L = _info.num_cores, _info.num_subcores, _info.num_lanes
_NW = _NC * _NS
_ROWS_PER_W = N // _NW


def _sc_gather_transpose(idx_flat, table):
    mesh = plsc.VectorSubcoreMesh(core_axis_name="c", subcore_axis_name="s")

    @functools.partial(
        pl.kernel,
        mesh=mesh,
        compiler_params=pltpu.CompilerParams(
            use_tc_tiling_on_sc=False, needs_layout_passes=False
        ),
        out_type=jax.ShapeDtypeStruct((NUM_HEADS, NN), jnp.float32),
        scratch_types=[
            pltpu.VMEM((N,), jnp.int32),
            pltpu.VMEM((N, NUM_HEADS), jnp.float32),
            pltpu.VMEM((NUM_HEADS, N), jnp.float32),
            pltpu.SemaphoreType.DMA,
        ],
    )
    def k(table_hbm, idx_hbm, out_hbm, idx_v, rows_v, trans_v, sem):
        wid = lax.axis_index("s") * _NC + lax.axis_index("c")
        row0 = wid * _ROWS_PER_W
        lane = lax.broadcasted_iota(jnp.int32, (_L,), 0)

        def row_body(r, carry):
            i = row0 + r
            pltpu.sync_copy(idx_hbm.at[pl.ds(i * N, N)], idx_v)
            pltpu.async_copy(table_hbm.at[idx_v], rows_v, sem).wait()

            def kb_body(kb, c2):
                base_k = kb * _L
                for j in range(_L):
                    kk = base_k + j
                    v = rows_v[kk]
                    plsc.store_scatter(
                        trans_v, [lane, jnp.full((_L,), kk, jnp.int32)], v
                    )
                return c2

            lax.fori_loop(0, N // _L, kb_body, 0)
            pltpu.sync_copy(trans_v, out_hbm.at[:, pl.ds(i * N, N)])
            return carry

        lax.fori_loop(0, _ROWS_PER_W, row_body, 0)

    return k(table, idx_flat)


_BR = 128


def _tc_body(g_ref, b_ref, out_ref):
    for tj in range(N // 128):
        out_ref[0, :, pl.ds(tj * 128, 128)] = (
            g_ref[0, :, tj, :] + b_ref[0, :, pl.ds(tj * 128, 128)]
        )


def _tc_assemble_add(g_raw, attn_bias):
    g4 = g_raw.reshape(NUM_HEADS, N, N // 128, 128)
    return pl.pallas_call(
        _tc_body,
        grid=(N // _BR, NUM_HEADS),
        in_specs=[
            pl.BlockSpec((1, _BR, N // 128, 128), lambda ib, h: (h, ib, 0, 0)),
            pl.BlockSpec((1, _BR, N), lambda ib, h: (0, ib, 0)),
        ],
        out_specs=pl.BlockSpec((1, _BR, N), lambda ib, h: (h, ib, 0)),
        out_shape=jax.ShapeDtypeStruct((NUM_HEADS, N, N), jnp.float32),
    )(g4, attn_bias)


def kernel(attn_bias, spatial_pos, W):
    idx_flat = spatial_pos.reshape(NN)
    g_raw = _sc_gather_transpose(idx_flat, W)
    out = _tc_assemble_add(g_raw, attn_bias)
    return out.reshape(1, NUM_HEADS, N, N)


# speedup vs baseline: 59.7197x; 1.9747x over previous
import functools

import jax
import jax.numpy as jnp
from jax import lax
from jax.experimental import pallas as pl
from jax.experimental.pallas import tpu as pltpu
from jax.experimental.pallas import tpu_sc as plsc

NUM_HEADS = 16
N = 2048
NN = N * N

_info = plsc.get_sparse_core_info()
_NC, _NS, _L = _info.num_cores, _info.num_subcores, _info.num_lanes
_NW = _NC * _NS
_B_PER_W = NN // _NW
_C = 1024
_CHUNKS = _B_PER_W // _C
_TPAD = _C + 1


def _sc_gather_transpose(idx_flat, table):
    mesh = plsc.VectorSubcoreMesh(core_axis_name="c", subcore_axis_name="s")

    @functools.partial(
        pl.kernel,
        mesh=mesh,
        compiler_params=pltpu.CompilerParams(
            use_tc_tiling_on_sc=False, needs_layout_passes=False
        ),
        out_type=jax.ShapeDtypeStruct((NUM_HEADS, NN), jnp.float32),
        scratch_types=[
            pltpu.VMEM((_C,), jnp.int32),
            pltpu.VMEM((_C,), jnp.int32),
            pltpu.VMEM((_C, NUM_HEADS), jnp.float32),
            pltpu.VMEM((_C, NUM_HEADS), jnp.float32),
            pltpu.VMEM((NUM_HEADS, _TPAD), jnp.float32),
            pltpu.VMEM((NUM_HEADS, _TPAD), jnp.float32),
            pltpu.SemaphoreType.DMA,
            pltpu.SemaphoreType.DMA,
            pltpu.SemaphoreType.DMA,
            pltpu.SemaphoreType.DMA,
        ],
    )
    def k(table_hbm, idx_hbm, out_hbm, i0, i1, r0, r1, t0, t1,
          g0, g1, w0, w1):
        idx_v = (i0, i1)
        rows_v = (r0, r1)
        trans_v = (t0, t1)
        gsem = (g0, g1)
        wsem = (w0, w1)
        wid = lax.axis_index("s") * _NC + lax.axis_index("c")
        wstart = wid * _B_PER_W
        lane = lax.broadcasted_iota(jnp.int32, (_L,), 0)

        pltpu.sync_copy(idx_hbm.at[pl.ds(wstart, _C)], idx_v[0])
        pltpu.async_copy(table_hbm.at[idx_v[0]], rows_v[0], gsem[0])

        def pair_body(tt, carry):
            for b in range(2):
                t = tt * 2 + b
                nb = (b + 1) % 2
                nxt = t + 1

                @pl.when(nxt < _CHUNKS)
                def _():
                    noff = wstart + nxt * _C
                    pltpu.sync_copy(idx_hbm.at[pl.ds(noff, _C)], idx_v[nb])
                    pltpu.async_copy(
                        table_hbm.at[idx_v[nb]], rows_v[nb], gsem[nb]
                    )

                pltpu.make_async_copy(
                    table_hbm.at[idx_v[b]], rows_v[b], gsem[b]
                ).wait()

                @pl.when(t >= 2)
                def _():
                    poff = wstart + (t - 2) * _C
                    pltpu.make_async_copy(
                        trans_v[b].at[:, pl.ds(0, _C)],
                        out_hbm.at[:, pl.ds(poff, _C)],
                        wsem[b],
                    ).wait()

                rref = rows_v[b]
                tref = trans_v[b]

                def kb_body(kb, col):
                    base_k = kb * _L
                    for j in range(_L):
                        v = rref[base_k + j]
                        plsc.store_scatter(tref, [lane, col], v)
                        col = col + 1
                    return col

                lax.fori_loop(
                    0, _C // _L, kb_body, jnp.zeros((_L,), jnp.int32)
                )

                off = wstart + t * _C
                pltpu.async_copy(
                    tref.at[:, pl.ds(0, _C)],
                    out_hbm.at[:, pl.ds(off, _C)],
                    wsem[b],
                )
            return carry

        lax.fori_loop(0, _CHUNKS // 2, pair_body, 0)

        for b in range(2):
            t = _CHUNKS - 2 + b
            off = wstart + t * _C
            pltpu.make_async_copy(
                trans_v[b].at[:, pl.ds(0, _C)],
                out_hbm.at[:, pl.ds(off, _C)],
                wsem[b],
            ).wait()

    return k(table, idx_flat)


_BR = 128


def _tc_body(g_ref, b_ref, out_ref):
    for tj in range(N // 128):
        out_ref[0, :, pl.ds(tj * 128, 128)] = (
            g_ref[0, :, tj, :] + b_ref[0, :, pl.ds(tj * 128, 128)]
        )


def _tc_assemble_add(g_raw, attn_bias):
    g4 = g_raw.reshape(NUM_HEADS, N, N // 128, 128)
    return pl.pallas_call(
        _tc_body,
        grid=(N // _BR, NUM_HEADS),
        in_specs=[
            pl.BlockSpec((1, _BR, N // 128, 128), lambda ib, h: (h, ib, 0, 0)),
            pl.BlockSpec((1, _BR, N), lambda ib, h: (0, ib, 0)),
        ],
        out_specs=pl.BlockSpec((1, _BR, N), lambda ib, h: (h, ib, 0)),
        out_shape=jax.ShapeDtypeStruct((NUM_HEADS, N, N), jnp.float32),
    )(g4, attn_bias)


def kernel(attn_bias, spatial_pos, W):
    idx_flat = spatial_pos.reshape(NN)
    g_raw = _sc_gather_transpose(idx_flat, W)
    out = _tc_assemble_add(g_raw, attn_bias)
    return out.reshape(1, NUM_HEADS, N, N)


# speedup vs baseline: 65.7198x; 1.1005x over previous
import functools

import jax
import jax.numpy as jnp
from jax import lax
from jax.experimental import pallas as pl
from jax.experimental.pallas import tpu as pltpu
from jax.experimental.pallas import tpu_sc as plsc

NUM_HEADS = 16
N = 2048
NN = N * N

_info = plsc.get_sparse_core_info()
_NC, _NS, _L = _info.num_cores, _info.num_subcores, _info.num_lanes
_NW = _NC * _NS
_B_PER_W = NN // _NW
_C = 1024
_CHUNKS = _B_PER_W // _C
_TPAD = _C + 1


def _sc_gather_transpose(idx_flat, table):
    mesh = plsc.VectorSubcoreMesh(core_axis_name="c", subcore_axis_name="s")

    @functools.partial(
        pl.kernel,
        mesh=mesh,
        compiler_params=pltpu.CompilerParams(
            use_tc_tiling_on_sc=False, needs_layout_passes=False
        ),
        out_type=jax.ShapeDtypeStruct((NUM_HEADS, NN), jnp.float32),
        scratch_types=[
            pltpu.VMEM((_C,), jnp.int32),
            pltpu.VMEM((_C,), jnp.int32),
            pltpu.VMEM((_C, NUM_HEADS), jnp.float32),
            pltpu.VMEM((_C, NUM_HEADS), jnp.float32),
            pltpu.VMEM((NUM_HEADS, _TPAD), jnp.float32),
            pltpu.VMEM((NUM_HEADS, _TPAD), jnp.float32),
            pltpu.SemaphoreType.DMA,
            pltpu.SemaphoreType.DMA,
            pltpu.SemaphoreType.DMA,
            pltpu.SemaphoreType.DMA,
        ],
    )
    def k(table_hbm, idx_hbm, out_hbm, i0, i1, r0, r1, t0, t1,
          g0, g1, w0, w1):
        idx_v = (i0, i1)
        rows_v = (r0, r1)
        trans_v = (t0, t1)
        gsem = (g0, g1)
        wsem = (w0, w1)
        wid = lax.axis_index("s") * _NC + lax.axis_index("c")
        wstart = wid * _B_PER_W
        lane = lax.broadcasted_iota(jnp.int32, (_L,), 0)

        pltpu.sync_copy(idx_hbm.at[pl.ds(wstart, _C)], idx_v[0])
        pltpu.async_copy(table_hbm.at[idx_v[0]], rows_v[0], gsem[0])

        def pair_body(tt, carry):
            for b in range(2):
                t = tt * 2 + b
                nb = (b + 1) % 2
                nxt = t + 1

                @pl.when(nxt < _CHUNKS)
                def _():
                    noff = wstart + nxt * _C
                    pltpu.sync_copy(idx_hbm.at[pl.ds(noff, _C)], idx_v[nb])
                    pltpu.async_copy(
                        table_hbm.at[idx_v[nb]], rows_v[nb], gsem[nb]
                    )

                pltpu.make_async_copy(
                    table_hbm.at[idx_v[b]], rows_v[b], gsem[b]
                ).wait()

                @pl.when(t >= 2)
                def _():
                    poff = wstart + (t - 2) * _C
                    pltpu.make_async_copy(
                        trans_v[b].at[:, pl.ds(0, _C)],
                        out_hbm.at[:, pl.ds(poff, _C)],
                        wsem[b],
                    ).wait()

                rref = rows_v[b]
                tref = trans_v[b]

                def kb_body(kb, carry):
                    base_k = kb * _L
                    colbase = jnp.full((_L,), base_k, jnp.int32)
                    for j in range(_L):
                        v = rref[base_k + j]
                        plsc.store_scatter(tref, [lane, colbase + j], v)
                    return carry

                lax.fori_loop(0, _C // _L, kb_body, 0)

                off = wstart + t * _C
                pltpu.async_copy(
                    tref.at[:, pl.ds(0, _C)],
                    out_hbm.at[:, pl.ds(off, _C)],
                    wsem[b],
                )
            return carry

        lax.fori_loop(0, _CHUNKS // 2, pair_body, 0)

        for b in range(2):
            t = _CHUNKS - 2 + b
            off = wstart + t * _C
            pltpu.make_async_copy(
                trans_v[b].at[:, pl.ds(0, _C)],
                out_hbm.at[:, pl.ds(off, _C)],
                wsem[b],
            ).wait()

    return k(table, idx_flat)


_TI = N // 8
_TJ = N // 128


def _tc_body(g_ref, b_ref, out_ref):
    out_ref[0] = jnp.reshape(g_ref[0, :, 0, :, :], (N, 128)) + b_ref[0]


def _tc_assemble_add(g_raw, attn_bias):
    g5 = g_raw.reshape(NUM_HEADS, _TI, _TJ, 8, 128)
    return pl.pallas_call(
        _tc_body,
        grid=(_TJ, NUM_HEADS),
        in_specs=[
            pl.BlockSpec((1, _TI, 1, 8, 128), lambda tj, h: (h, 0, tj, 0, 0)),
            pl.BlockSpec((1, N, 128), lambda tj, h: (0, 0, tj)),
        ],
        out_specs=pl.BlockSpec((1, N, 128), lambda tj, h: (h, 0, tj)),
        out_shape=jax.ShapeDtypeStruct((NUM_HEADS, N, N), jnp.float32),
    )(g5, attn_bias)


def kernel(attn_bias, spatial_pos, W):
    idx_tile = (
        spatial_pos.reshape(_TI, 8, _TJ, 128)
        .transpose(0, 2, 1, 3)
        .reshape(NN)
    )
    g_raw = _sc_gather_transpose(idx_tile, W)
    out = _tc_assemble_add(g_raw, attn_bias)
    return out.reshape(1, NUM_HEADS, N, N)


# speedup vs baseline: 73.1951x; 1.1137x over previous
import functools

import jax
import jax.numpy as jnp
from jax import lax
from jax.experimental import pallas as pl
from jax.experimental.pallas import tpu as pltpu
from jax.experimental.pallas import tpu_sc as plsc

NUM_HEADS = 16
N = 2048
NN = N * N

_info = plsc.get_sparse_core_info()
_NC, _NS, _L = _info.num_cores, _info.num_subcores, _info.num_lanes
_NW = _NC * _NS
_B_PER_W = NN // _NW
_C = 1024
_CHUNKS = _B_PER_W // _C
_TPAD = _C + 1
_CPB = 16
_IBC = _CPB * _C
_NBLK = _CHUNKS // _CPB


def _sc_gather_transpose(idx_flat, table):
    mesh = plsc.VectorSubcoreMesh(core_axis_name="c", subcore_axis_name="s")

    @functools.partial(
        pl.kernel,
        mesh=mesh,
        compiler_params=pltpu.CompilerParams(
            use_tc_tiling_on_sc=False, needs_layout_passes=False
        ),
        out_type=jax.ShapeDtypeStruct((NUM_HEADS, NN), jnp.float32),
        scratch_types=[
            pltpu.VMEM((_C,), jnp.int32),
            pltpu.VMEM((_C,), jnp.int32),
            pltpu.VMEM((_C, NUM_HEADS), jnp.float32),
            pltpu.VMEM((_C, NUM_HEADS), jnp.float32),
            pltpu.VMEM((NUM_HEADS, _TPAD), jnp.float32),
            pltpu.VMEM((NUM_HEADS, _TPAD), jnp.float32),
            pltpu.SemaphoreType.DMA,
            pltpu.SemaphoreType.DMA,
            pltpu.SemaphoreType.DMA,
            pltpu.SemaphoreType.DMA,
            pltpu.SemaphoreType.DMA,
            pltpu.SemaphoreType.DMA,
        ],
    )
    def k(table_hbm, idx_hbm, out_hbm, i0, i1, r0, r1, t0, t1,
          is0, is1, g0, g1, w0, w1):
        idx_v = (i0, i1)
        rows_v = (r0, r1)
        trans_v = (t0, t1)
        isem = (is0, is1)
        gsem = (g0, g1)
        wsem = (w0, w1)
        wid = lax.axis_index("s") * _NC + lax.axis_index("c")
        wstart = wid * _B_PER_W
        lane = lax.broadcasted_iota(jnp.int32, (_L,), 0)

        def idx_desc(t, p):
            return pltpu.make_async_copy(
                idx_hbm.at[pl.ds(wstart + t * _C, _C)], idx_v[p], isem[p]
            )

        def gather_desc(p):
            return pltpu.make_async_copy(
                table_hbm.at[idx_v[p]], rows_v[p], gsem[p]
            )

        def wb_desc(t, p):
            return pltpu.make_async_copy(
                trans_v[p].at[:, pl.ds(0, _C)],
                out_hbm.at[:, pl.ds(wstart + t * _C, _C)],
                wsem[p],
            )

        def compute_chunk(p):
            rref = rows_v[p]
            tref = trans_v[p]

            def kb_body(kb, c3):
                base_k = kb * _L
                colbase = jnp.full((_L,), base_k, jnp.int32)
                for j in range(_L):
                    v = rref[base_k + j]
                    plsc.store_scatter(tref, [lane, colbase + j], v)
                return c3

            lax.fori_loop(0, _C // _L, kb_body, 0)

        idx_desc(0, 0).start()
        idx_desc(1, 1).start()
        idx_desc(0, 0).wait()
        gather_desc(0).start()

        def pair_body(tt, carry):
            for cb in range(2):
                t = tt * 2 + cb
                nb = (cb + 1) % 2

                gather_desc(cb).wait()

                @pl.when(t + 2 < _CHUNKS)
                def _():
                    idx_desc(t + 2, cb).start()

                @pl.when(t + 1 < _CHUNKS)
                def _():
                    idx_desc(t + 1, nb).wait()
                    gather_desc(nb).start()

                @pl.when(t >= 2)
                def _():
                    wb_desc(t - 2, cb).wait()

                compute_chunk(cb)
                wb_desc(t, cb).start()
            return carry

        lax.fori_loop(0, _CHUNKS // 2, pair_body, 0)

        for b in range(2):
            wb_desc(_CHUNKS - 2 + b, b).wait()

    return k(table, idx_flat)


_TI = N // 8
_TJ = N // 128


def _tc_body(g_ref, b_ref, out_ref):
    out_ref[0] = jnp.reshape(g_ref[0, :, 0, :, :], (N, 128)) + b_ref[0]


def _tc_assemble_add(g_raw, attn_bias):
    g5 = g_raw.reshape(NUM_HEADS, _TI, _TJ, 8, 128)
    return pl.pallas_call(
        _tc_body,
        grid=(_TJ, NUM_HEADS),
        in_specs=[
            pl.BlockSpec((1, _TI, 1, 8, 128), lambda tj, h: (h, 0, tj, 0, 0)),
            pl.BlockSpec((1, N, 128), lambda tj, h: (0, 0, tj)),
        ],
        out_specs=pl.BlockSpec((1, N, 128), lambda tj, h: (h, 0, tj)),
        out_shape=jax.ShapeDtypeStruct((NUM_HEADS, N, N), jnp.float32),
    )(g5, attn_bias)


def kernel(attn_bias, spatial_pos, W):
    idx_tile = (
        spatial_pos.reshape(_TI, 8, _TJ, 128)
        .transpose(0, 2, 1, 3)
        .reshape(NN)
    )
    g_raw = _sc_gather_transpose(idx_tile, W)
    out = _tc_assemble_add(g_raw, attn_bias)
    return out.reshape(1, NUM_HEADS, N, N)
